# baseline (speedup 1.0000x reference)
import functools
import jax
import jax.numpy as jnp
from jax import lax
from jax.experimental import pallas as pl
from jax.experimental.pallas import tpu as pltpu
from jax.experimental.pallas import tpu_sc as plsc

_B, _C, _T = 64, 64, 80
_EPS = 1e-9
_NW = 32
_BPW = _B // _NW
_CT = _C * _T


def _newton_sqrt(s):
    u = lax.bitcast_convert_type(s, jnp.int32)
    u = 0x5F3759DF - (u >> 1)
    r = lax.bitcast_convert_type(u, jnp.float32)
    hs = 0.5 * s
    r = r * (1.5 - hs * r * r)
    r = r * (1.5 - hs * r * r)
    return s * r


def _sc_diversity(x, y):
    mesh = plsc.VectorSubcoreMesh(core_axis_name="c", subcore_axis_name="s")

    @functools.partial(
        pl.kernel,
        mesh=mesh,
        out_type=jax.ShapeDtypeStruct((_NW, 16), jnp.float32),
        scratch_types=[
            pltpu.VMEM((_CT,), jnp.float32),
            pltpu.VMEM((_CT,), jnp.float32),
            pltpu.VMEM((16,), jnp.float32),
            pltpu.SemaphoreType.DMA,
        ],
    )
    def sck(x_hbm, y_hbm, out_hbm, xv, yv, accv, sem):
        wid = lax.axis_index("s") * 2 + lax.axis_index("c")
        acc = jnp.zeros((16,), jnp.float32)
        for bi in range(_BPW):
            batch = wid * _BPW + bi
            pltpu.async_copy(x_hbm.at[batch], xv, sem).wait()
            pltpu.async_copy(y_hbm.at[batch], yv, sem).wait()

            def k_body(k, acc_k):
                def r_body(r, acc_r):
                    r2 = jnp.where(r + k < _C, r + k, r + k - _C)
                    o1 = r * _T
                    o2 = r2 * _T
                    a = acc_r
                    for tc in range(_T // 16):
                        dx = (xv[pl.ds(o1 + 16 * tc, 16)]
                              - xv[pl.ds(o2 + 16 * tc, 16)])
                        dy = (yv[pl.ds(o1 + 16 * tc, 16)]
                              - yv[pl.ds(o2 + 16 * tc, 16)])
                        s = dx * dx + dy * dy + _EPS
                        a = a + _newton_sqrt(s)
                    return a

                return lax.fori_loop(0, _C, r_body, acc_k)

            acc = lax.fori_loop(1, _C // 2, k_body, acc)

            def r32_body(r, acc_r):
                r2 = jnp.where(r + _C // 2 < _C, r + _C // 2, r - _C // 2)
                o1 = r * _T
                o2 = r2 * _T
                a = acc_r
                for tc in range(_T // 16):
                    dx = (xv[pl.ds(o1 + 16 * tc, 16)]
                          - xv[pl.ds(o2 + 16 * tc, 16)])
                    dy = (yv[pl.ds(o1 + 16 * tc, 16)]
                          - yv[pl.ds(o2 + 16 * tc, 16)])
                    s = dx * dx + dy * dy + _EPS
                    a = a + 0.5 * _newton_sqrt(s)
                return a

            acc = lax.fori_loop(0, _C, r32_body, acc)
        accv[...] = acc
        pltpu.async_copy(accv, out_hbm.at[wid], sem).wait()

    return sck(x, y)


def kernel(predicted_trajectory):
    traj = predicted_trajectory[..., :2]
    x = traj[..., 0].reshape(_B, _CT)
    y = traj[..., 1].reshape(_B, _CT)
    partials = _sc_diversity(x, y)
    total = 2.0 * jnp.sum(partials)
    scale = -1.0 / (_T * _C * (_C - 1) * _B)
    return total * scale
